# baseline (device time: 262337 ns/iter reference)
import jax
import jax.numpy as jnp
from jax import lax
from jax.experimental import pallas as pl
from jax.experimental.pallas import tpu as pltpu

T = 4096
TOK_HALF = T // 2
NBUF = 32
SIZES = (32, 32, 64) + (128,) * 15
EDGES = tuple(sum(SIZES[:c]) for c in range(len(SIZES)))
C = len(SIZES)


def kernel(ids, E):
    v_shard, d = E.shape
    my_x = lax.axis_index("x")
    my_y = lax.axis_index("y")

    ids_half = lax.dynamic_slice(ids, (my_x * TOK_HALF,), (TOK_HALF,))
    local = ids_half - my_y * v_shard
    valid = (local >= 0) & (local < v_shard)
    safe = jnp.where(valid, local, 0).astype(jnp.int32)
    mask = valid.astype(jnp.float32)[:, None]

    hi = jax.lax.Precision.HIGHEST

    def compact(v, rows, pos0, batch):
        s = v.shape[0] // batch
        v2d = v.reshape(batch, s)
        vi = v2d.astype(jnp.int32)
        within = jnp.cumsum(vi, axis=1) - vi
        cnt_b = jnp.sum(vi, axis=1).astype(jnp.int32)
        onehot = (
            v2d[:, None, :]
            & (within[:, None, :] == jnp.arange(s, dtype=jnp.int32)[None, :, None])
        ).astype(jnp.float32)
        rows_f = rows.reshape(batch, s).astype(jnp.float32)
        pos_f = (
            pos0 + jnp.arange(batch * s, dtype=jnp.float32)
        ).reshape(batch, s)
        r = jnp.einsum("cki,ci->ck", onehot, rows_f, precision=hi)
        p = jnp.einsum("cki,ci->ck", onehot, pos_f, precision=hi)
        return (
            r.astype(jnp.int32).reshape(-1),
            p.astype(jnp.int32).reshape(-1),
            cnt_b,
        )

    parts_r, parts_p, parts_c = [], [], []
    for e0, s in zip(EDGES[:3], SIZES[:3]):
        r, p, cb = compact(valid[e0:e0 + s], safe[e0:e0 + s], float(e0), 1)
        parts_r.append(r); parts_p.append(p); parts_c.append(cb)
    head = EDGES[3]
    r, p, cb = compact(valid[head:], safe[head:], float(head), C - 3)
    parts_r.append(r); parts_p.append(p); parts_c.append(cb)
    ow_row = jnp.concatenate(parts_r)
    ow_pos = jnp.concatenate(parts_p)
    cnt = jnp.concatenate(parts_c)

    def body(ow_row_ref, ow_pos_ref, cnt_ref, mask_ref, E_ref, out_ref,
             gbuf_ref, rbuf_ref, gsems, csems,
             s1send, s1recv, s2send, s2recv):
        x = lax.axis_index("x")
        y = lax.axis_index("y")
        off = x * TOK_HALF

        barrier = pltpu.get_barrier_semaphore()
        pl.semaphore_signal(barrier, inc=1, device_id=(x, 1 - y),
                            device_id_type=pl.DeviceIdType.MESH)
        pl.semaphore_signal(barrier, inc=1, device_id=(1 - x, y),
                            device_id_type=pl.DeviceIdType.MESH)

        def wait_slot(slot):
            pltpu.make_async_copy(
                E_ref.at[pl.ds(0, 1), :],
                gbuf_ref.at[pl.ds(0, 1), :],
                gsems.at[slot],
            ).wait()

        def gather_chunk(c):
            base = EDGES[c]
            n = cnt_ref[c]

            def step(k, t):
                slot = lax.rem(k, NBUF)

                @pl.when(k >= NBUF)
                def _():
                    wait_slot(slot)
                pltpu.make_async_copy(
                    E_ref.at[pl.ds(ow_row_ref[base + k], 1), :],
                    gbuf_ref.at[pl.ds(ow_pos_ref[base + k], 1), :],
                    gsems.at[slot],
                ).start()
                return t
            lax.fori_loop(0, n, step, 0)

            def drain(k, t):
                wait_slot(lax.rem(k, NBUF))
                return t
            lax.fori_loop(jnp.maximum(n - NBUF, 0), n, drain, 0)

        def rdma1(c):
            return pltpu.make_async_remote_copy(
                src_ref=gbuf_ref.at[pl.ds(EDGES[c], SIZES[c])],
                dst_ref=rbuf_ref.at[pl.ds(EDGES[c], SIZES[c])],
                send_sem=s1send.at[c],
                recv_sem=s1recv.at[c],
                device_id=(x, 1 - y),
                device_id_type=pl.DeviceIdType.MESH,
            )

        def rdma2(c):
            return pltpu.make_async_remote_copy(
                src_ref=rbuf_ref.at[pl.ds(EDGES[c], SIZES[c])],
                dst_ref=out_ref.at[pl.ds(off + EDGES[c], SIZES[c])],
                send_sem=s2send.at[c],
                recv_sem=s2recv.at[c],
                device_id=(1 - x, y),
                device_id_type=pl.DeviceIdType.MESH,
            )

        gather_chunk(0)
        pl.semaphore_wait(barrier, 2)
        rdma1(0).start()
        for c in range(1, C):
            gather_chunk(c)
            rdma1(c).start()

        for c in range(C):
            rdma1(c).wait_recv()
            e0, e1 = EDGES[c], EDGES[c] + SIZES[c]
            sl = pl.ds(EDGES[c], SIZES[c])
            rbuf_ref[sl, :] = jnp.where(
                mask_ref[e0:e1, :] != 0.0,
                gbuf_ref[e0:e1, :],
                rbuf_ref[sl, :],
            )
            rdma2(c).start()
            pltpu.make_async_copy(
                rbuf_ref.at[sl],
                out_ref.at[pl.ds(off + EDGES[c], SIZES[c])],
                csems.at[c],
            ).start()

        for c in range(C):
            rdma2(c).wait_recv()
        for c in range(C):
            pltpu.make_async_copy(
                rbuf_ref.at[pl.ds(EDGES[c], SIZES[c])],
                out_ref.at[pl.ds(off + EDGES[c], SIZES[c])],
                csems.at[c],
            ).wait()
            rdma1(c).wait_send()
            rdma2(c).wait_send()

    return pl.pallas_call(
        body,
        out_shape=jax.ShapeDtypeStruct((T, d), jnp.float32),
        in_specs=[
            pl.BlockSpec(memory_space=pltpu.SMEM),
            pl.BlockSpec(memory_space=pltpu.SMEM),
            pl.BlockSpec(memory_space=pltpu.SMEM),
            pl.BlockSpec(memory_space=pltpu.VMEM),
            pl.BlockSpec(memory_space=pl.ANY),
        ],
        out_specs=pl.BlockSpec(memory_space=pl.ANY),
        scratch_shapes=[
            pltpu.VMEM((TOK_HALF, d), jnp.float32),
            pltpu.VMEM((TOK_HALF, d), jnp.float32),
            pltpu.SemaphoreType.DMA((NBUF,)),
            pltpu.SemaphoreType.DMA((C,)),
            pltpu.SemaphoreType.DMA((C,)),
            pltpu.SemaphoreType.DMA((C,)),
            pltpu.SemaphoreType.DMA((C,)),
            pltpu.SemaphoreType.DMA((C,)),
        ],
        compiler_params=pltpu.CompilerParams(
            collective_id=0,
            vmem_limit_bytes=100 * 1024 * 1024,
        ),
    )(ow_row, ow_pos, cnt, mask, E)


# device time: 254634 ns/iter; 1.0303x vs baseline; 1.0303x over previous
import jax
import jax.numpy as jnp
from jax import lax
from jax.experimental import pallas as pl
from jax.experimental.pallas import tpu as pltpu

T = 4096
TOK_HALF = T // 2
NBUF = 32
SIZES = (128,) * 16
EDGES = tuple(sum(SIZES[:c]) for c in range(len(SIZES)))
C = len(SIZES)


def kernel(ids, E):
    v_shard, d = E.shape
    my_x = lax.axis_index("x")
    my_y = lax.axis_index("y")

    ids_half = lax.dynamic_slice(ids, (my_x * TOK_HALF,), (TOK_HALF,))
    local = ids_half - my_y * v_shard
    valid = (local >= 0) & (local < v_shard)
    safe = jnp.where(valid, local, 0).astype(jnp.int32)
    mask = valid.astype(jnp.float32)[:, None]

    hi = jax.lax.Precision.HIGHEST

    def compact(v, rows, pos0, batch):
        s = v.shape[0] // batch
        v2d = v.reshape(batch, s)
        vi = v2d.astype(jnp.int32)
        within = jnp.cumsum(vi, axis=1) - vi
        cnt_b = jnp.sum(vi, axis=1).astype(jnp.int32)
        onehot = (
            v2d[:, None, :]
            & (within[:, None, :] == jnp.arange(s, dtype=jnp.int32)[None, :, None])
        ).astype(jnp.float32)
        rows_f = rows.reshape(batch, s).astype(jnp.float32)
        pos_f = (
            pos0 + jnp.arange(batch * s, dtype=jnp.float32)
        ).reshape(batch, s)
        r = jnp.einsum("cki,ci->ck", onehot, rows_f, precision=hi)
        p = jnp.einsum("cki,ci->ck", onehot, pos_f, precision=hi)
        return (
            r.astype(jnp.int32).reshape(-1),
            p.astype(jnp.int32).reshape(-1),
            cnt_b,
        )

    ow_row, ow_pos, cnt = compact(valid, safe, 0.0, C)

    def body(ow_row_ref, ow_pos_ref, cnt_ref, mask_ref, E_ref, out_ref,
             gbuf_ref, rbuf_ref, gsems, csems,
             s1send, s1recv, s2send, s2recv):
        x = lax.axis_index("x")
        y = lax.axis_index("y")
        off = x * TOK_HALF

        barrier = pltpu.get_barrier_semaphore()
        pl.semaphore_signal(barrier, inc=1, device_id=(x, 1 - y),
                            device_id_type=pl.DeviceIdType.MESH)
        pl.semaphore_signal(barrier, inc=1, device_id=(1 - x, y),
                            device_id_type=pl.DeviceIdType.MESH)

        def wait_slot(slot):
            pltpu.make_async_copy(
                E_ref.at[pl.ds(0, 1), :],
                gbuf_ref.at[pl.ds(0, 1), :],
                gsems.at[slot],
            ).wait()

        def gather_chunk(c):
            base = EDGES[c]
            n = cnt_ref[c]

            def step(k, t):
                slot = lax.rem(k, NBUF)

                @pl.when(k >= NBUF)
                def _():
                    wait_slot(slot)
                pltpu.make_async_copy(
                    E_ref.at[pl.ds(ow_row_ref[base + k], 1), :],
                    gbuf_ref.at[pl.ds(ow_pos_ref[base + k], 1), :],
                    gsems.at[slot],
                ).start()
                return t
            lax.fori_loop(0, n, step, 0)

            def drain(k, t):
                wait_slot(lax.rem(k, NBUF))
                return t
            lax.fori_loop(jnp.maximum(n - NBUF, 0), n, drain, 0)

        def rdma1(c):
            return pltpu.make_async_remote_copy(
                src_ref=gbuf_ref.at[pl.ds(EDGES[c], SIZES[c])],
                dst_ref=rbuf_ref.at[pl.ds(EDGES[c], SIZES[c])],
                send_sem=s1send.at[c],
                recv_sem=s1recv.at[c],
                device_id=(x, 1 - y),
                device_id_type=pl.DeviceIdType.MESH,
            )

        def rdma2(c):
            return pltpu.make_async_remote_copy(
                src_ref=rbuf_ref.at[pl.ds(EDGES[c], SIZES[c])],
                dst_ref=out_ref.at[pl.ds(off + EDGES[c], SIZES[c])],
                send_sem=s2send.at[c],
                recv_sem=s2recv.at[c],
                device_id=(1 - x, y),
                device_id_type=pl.DeviceIdType.MESH,
            )

        gather_chunk(0)
        pl.semaphore_wait(barrier, 2)
        rdma1(0).start()
        for c in range(1, C):
            gather_chunk(c)
            rdma1(c).start()

        for c in range(C):
            rdma1(c).wait_recv()
            e0, e1 = EDGES[c], EDGES[c] + SIZES[c]
            sl = pl.ds(EDGES[c], SIZES[c])
            rbuf_ref[sl, :] = jnp.where(
                mask_ref[e0:e1, :] != 0.0,
                gbuf_ref[e0:e1, :],
                rbuf_ref[sl, :],
            )
            rdma2(c).start()
            pltpu.make_async_copy(
                rbuf_ref.at[sl],
                out_ref.at[pl.ds(off + EDGES[c], SIZES[c])],
                csems.at[c],
            ).start()

        for c in range(C):
            rdma2(c).wait_recv()
        for c in range(C):
            pltpu.make_async_copy(
                rbuf_ref.at[pl.ds(EDGES[c], SIZES[c])],
                out_ref.at[pl.ds(off + EDGES[c], SIZES[c])],
                csems.at[c],
            ).wait()
            rdma1(c).wait_send()
            rdma2(c).wait_send()

    return pl.pallas_call(
        body,
        out_shape=jax.ShapeDtypeStruct((T, d), jnp.float32),
        in_specs=[
            pl.BlockSpec(memory_space=pltpu.SMEM),
            pl.BlockSpec(memory_space=pltpu.SMEM),
            pl.BlockSpec(memory_space=pltpu.SMEM),
            pl.BlockSpec(memory_space=pltpu.VMEM),
            pl.BlockSpec(memory_space=pl.ANY),
        ],
        out_specs=pl.BlockSpec(memory_space=pl.ANY),
        scratch_shapes=[
            pltpu.VMEM((TOK_HALF, d), jnp.float32),
            pltpu.VMEM((TOK_HALF, d), jnp.float32),
            pltpu.SemaphoreType.DMA((NBUF,)),
            pltpu.SemaphoreType.DMA((C,)),
            pltpu.SemaphoreType.DMA((C,)),
            pltpu.SemaphoreType.DMA((C,)),
            pltpu.SemaphoreType.DMA((C,)),
            pltpu.SemaphoreType.DMA((C,)),
        ],
        compiler_params=pltpu.CompilerParams(
            collective_id=0,
            vmem_limit_bytes=100 * 1024 * 1024,
        ),
    )(ow_row, ow_pos, cnt, mask, E)
